# baseline (speedup 1.0000x reference)
import functools

import jax
import jax.numpy as jnp
from jax import lax
from jax.experimental import pallas as pl
from jax.experimental.pallas import tpu as pltpu
from jax.experimental.pallas import tpu_sc as plsc

_NUM_CORES = 2
_NUM_SUBCORES = 16
_NW = _NUM_CORES * _NUM_SUBCORES
_CHUNK = 128
_NBUF = 8
_DEPTH = 6


def _gather_flat(indices_2d, table):
    n_rows, C = indices_2d.shape
    D = table.shape[1]
    B = n_rows * C
    assert n_rows % _NW == 0
    n_chunks = n_rows // _NW
    b_per_w = n_chunks * C
    assert (n_chunks - (_NBUF - _DEPTH) - _DEPTH) % _NBUF == 0

    mesh = plsc.VectorSubcoreMesh(core_axis_name="c", subcore_axis_name="s")

    @functools.partial(
        pl.kernel,
        mesh=mesh,
        out_type=jax.ShapeDtypeStruct((B, D), jnp.float32),
        scratch_types=[
            pltpu.VMEM((n_chunks, C), jnp.int32),
            pltpu.VMEM((_NBUF, C, D), jnp.float32),
        ] + [pltpu.SemaphoreType.DMA] * _NBUF,
        compiler_params=pltpu.CompilerParams(use_tc_tiling_on_sc=False),
    )
    def k(idx_hbm, table_hbm, out_hbm, idx_v, rows_v, *sems):
        wid = lax.axis_index("s") * _NUM_CORES + lax.axis_index("c")
        base = wid * b_per_w

        pltpu.sync_copy(idx_hbm.at[pl.ds(wid * n_chunks, n_chunks)], idx_v)

        def start_gather(i, b):
            pltpu.async_copy(table_hbm.at[idx_v.at[i]], rows_v.at[b], sems[b])

        def wait_gather(i, b):
            pltpu.make_async_copy(
                table_hbm.at[idx_v.at[i]], rows_v.at[b], sems[b]
            ).wait()

        def start_store(i, b):
            pltpu.async_copy(
                rows_v.at[b], out_hbm.at[pl.ds(base + i * C, C)], sems[b]
            )

        def wait_store(i, b):
            pltpu.make_async_copy(
                rows_v.at[b], out_hbm.at[pl.ds(base + i * C, C)], sems[b]
            ).wait()

        for i in range(_DEPTH):
            start_gather(i, i)

        for i in range(_NBUF - _DEPTH):
            wait_gather(i, i)
            start_store(i, i)
            start_gather(i + _DEPTH, (i + _DEPTH) % _NBUF)

        lo = _NBUF - _DEPTH

        def body(j, carry):
            for u in range(_NBUF):
                i = lo + j * _NBUF + u
                b = (lo + u) % _NBUF
                bg = (lo + u + _DEPTH) % _NBUF
                wait_gather(i, b)
                start_store(i, b)
                wait_store(i + _DEPTH - _NBUF, bg)
                start_gather(i + _DEPTH, bg)
            return carry

        n_steady = (n_chunks - lo - _DEPTH) // _NBUF
        lax.fori_loop(0, n_steady, body, 0)

        for u in range(_DEPTH):
            i = n_chunks - _DEPTH + u
            wait_gather(i, i % _NBUF)
            start_store(i, i % _NBUF)

        for u in range(_NBUF):
            i = n_chunks - _NBUF + u
            wait_store(i, i % _NBUF)

    return k(indices_2d, table)


def kernel(indices, table):
    shape = indices.shape
    flat = indices.reshape(-1, _CHUNK).astype(jnp.int32)
    out = _gather_flat(flat, table)
    return out.reshape(*shape, table.shape[1])
